# baseline (speedup 1.0000x reference)
import functools

import jax
import jax.numpy as jnp
from jax import lax
from jax.experimental import pallas as pl
from jax.experimental.pallas import tpu as pltpu
from jax.experimental.pallas import tpu_sc as plsc

_N = 1000
_C = 2
_FEAT = 1024
_G = 16
_GEO = 64
_DG = 64
_K = 10
_NB = 8
_MC = 1000


def _proj_body(f_ref, wqw_ref, wqb_ref, wkw_ref, wkb_ref, cw_ref,
               wq_ref, wk_ref, fp_ref):
    fc = f_ref[0]
    dn = (((1,), (1,)), ((), ()))
    wq = lax.dot_general(fc, wqw_ref[...], dn, preferred_element_type=jnp.float32)
    wq_ref[0] = wq + wqb_ref[...]
    wk = lax.dot_general(fc, wkw_ref[...], dn, preferred_element_type=jnp.float32)
    wk_ref[0] = wk + wkb_ref[...]
    fp_ref[0] = lax.dot_general(fc, cw_ref[...], dn, preferred_element_type=jnp.float32)


def _proj_call(f_p, WQ_w, WQ_b, WK_w, WK_b, conv_w):
    full = lambda *shape: pl.BlockSpec(shape, lambda c: (0,) * len(shape))
    return pl.pallas_call(
        _proj_body,
        grid=(_C,),
        in_specs=[
            pl.BlockSpec((1, _N, _FEAT), lambda c: (c, 0, 0)),
            full(_FEAT, _FEAT),
            full(1, _FEAT),
            full(_FEAT, _FEAT),
            full(1, _FEAT),
            full(_FEAT, _FEAT),
        ],
        out_specs=[
            pl.BlockSpec((1, _N, _FEAT), lambda c: (c, 0, 0)),
            pl.BlockSpec((1, _N, _FEAT), lambda c: (c, 0, 0)),
            pl.BlockSpec((1, _N, _FEAT), lambda c: (c, 0, 0)),
        ],
        out_shape=[
            jax.ShapeDtypeStruct((_C, _N, _FEAT), jnp.float32),
            jax.ShapeDtypeStruct((_C, _N, _FEAT), jnp.float32),
            jax.ShapeDtypeStruct((_C, _N, _FEAT), jnp.float32),
        ],
        compiler_params=pltpu.CompilerParams(vmem_limit_bytes=100 * 2**20),
    )(f_p, WQ_w, WQ_b.reshape(1, _FEAT), WK_w, WK_b.reshape(1, _FEAT), conv_w)


_NBL = 5
_NG = _N // 8


def _attn_body(pe_ref, wq_ref, wk_ref, wgt_ref, b8_ref, spr_ref, msk_ref,
               soft_ref, idx_ref):
    wk = wk_ref[0]
    row = lax.broadcasted_iota(jnp.int32, (_N, 128), 0)
    big = jnp.int32(2**30)
    for i in range(_NBL):
        P = jnp.concatenate(
            [jnp.dot(pe_ref[0, i * 8 + nl], wgt_ref[...],
                     preferred_element_type=jnp.float32) for nl in range(8)],
            axis=1)
        L = jnp.log(jnp.maximum(P + b8_ref[...], 1e-6))
        wqb = wq_ref[0, i * 8:(i + 1) * 8, :]
        wqS = jnp.dot(jnp.transpose(wqb), spr_ref[...],
                      preferred_element_type=jnp.float32) * msk_ref[...]
        A = jnp.dot(wk, wqS, preferred_element_type=jnp.float32) * 0.125
        X = A + L
        vals, idxs = [], []
        for _ in range(_K):
            mx = jnp.max(X, axis=0, keepdims=True)
            am = jnp.min(jnp.where(X == mx, row, big), axis=0, keepdims=True)
            vals.append(mx)
            idxs.append(am)
            X = jnp.where(row == am, -jnp.inf, X)
        V = jnp.concatenate(vals, axis=0)
        E = jnp.exp(V - V[0:1])
        S = E / jnp.sum(E, axis=0, keepdims=True)
        soft_ref[0, i] = jnp.concatenate(
            [S, jnp.zeros((16 - _K, 128), jnp.float32)], axis=0)
        idx_ref[0, i] = jnp.concatenate(
            idxs + [jnp.zeros((16 - _K, 128), jnp.int32)], axis=0)


def _attn_call(pe, wq, wk, wgt, b8, spread, msk):
    return pl.pallas_call(
        _attn_body,
        grid=(_C, _NG // _NBL),
        in_specs=[
            pl.BlockSpec((1, 8 * _NBL, _N, _GEO), lambda c, nb: (c, nb, 0, 0)),
            pl.BlockSpec((1, 8 * _NBL, _FEAT), lambda c, nb: (c, nb, 0)),
            pl.BlockSpec((1, _N, _FEAT), lambda c, nb: (c, 0, 0)),
            pl.BlockSpec((_GEO, _G), lambda c, nb: (0, 0)),
            pl.BlockSpec((1, 128), lambda c, nb: (0, 0)),
            pl.BlockSpec((8, 128), lambda c, nb: (0, 0)),
            pl.BlockSpec((_FEAT, 128), lambda c, nb: (0, 0)),
        ],
        out_specs=[
            pl.BlockSpec((1, _NBL, 16, 128), lambda c, nb: (c, nb, 0, 0)),
            pl.BlockSpec((1, _NBL, 16, 128), lambda c, nb: (c, nb, 0, 0)),
        ],
        out_shape=[
            jax.ShapeDtypeStruct((_C, _NG, 16, 128), jnp.float32),
            jax.ShapeDtypeStruct((_C, _NG, 16, 128), jnp.int32),
        ],
        compiler_params=pltpu.CompilerParams(vmem_limit_bytes=100 * 2**20),
    )(pe, wq, wk, wgt, b8, spread, msk)


_CH = 8


def _sc_body(soft_hbm, idx_hbm, fp_hbm, cb_hbm, out_hbm,
             soft_v, idx_v, rid_v, rows_v, cb_v, outc_v, sem):
    nc = 2
    b = lax.axis_index("s") * nc + lax.axis_index("c")
    c = b // _G
    g = b % _G
    base = b * _N

    pltpu.sync_copy(soft_hbm.at[pl.ds(base, _N)], soft_v)
    pltpu.sync_copy(idx_hbm.at[pl.ds(base, _N)], idx_v)
    pltpu.sync_copy(cb_hbm.at[b], cb_v)

    off = c * (_G * _N) + g

    def rbody(i, carry):
        rid_v[i // _CH, pl.ds((i % _CH) * 16, 16)] = idx_v[i] * 16 + off
        return carry

    lax.fori_loop(0, _N, rbody, 0)

    def cbody(ch, carry):
        pltpu.async_copy(fp_hbm.at[rid_v.at[ch]], rows_v, sem).wait()
        for j in range(_CH):
            it = ch * _CH + j
            isp = jnp.full((16,), it, jnp.int32)
            accs = [cb_v[q * 16:(q + 1) * 16] for q in range(4)]
            for k in range(_K):
                ksp = jnp.full((16,), k, jnp.int32)
                w = plsc.load_gather(soft_v, [isp, ksp])
                r = j * 16 + k
                for q in range(4):
                    accs[q] = accs[q] + w * rows_v[r, q * 16:(q + 1) * 16]
            for q in range(4):
                outc_v[j, q * 16:(q + 1) * 16] = accs[q]
        pltpu.sync_copy(outc_v, out_hbm.at[pl.ds(base + ch * _CH, _CH)])
        return carry

    lax.fori_loop(0, _N // _CH, cbody, 0)


def _sc_call(soft2, idx2, fp_sc, cb32):
    mesh = plsc.VectorSubcoreMesh(core_axis_name="c", subcore_axis_name="s")
    fn = pl.kernel(
        _sc_body,
        out_type=jax.ShapeDtypeStruct((_C * _G * _N, _DG), jnp.float32),
        mesh=mesh,
        scratch_types=[
            pltpu.VMEM((_N, 16), jnp.float32),
            pltpu.VMEM((_N, 16), jnp.int32),
            pltpu.VMEM((_N // _CH, _CH * 16), jnp.int32),
            pltpu.VMEM((_CH * 16, _DG), jnp.float32),
            pltpu.VMEM((_DG,), jnp.float32),
            pltpu.VMEM((_CH, _DG), jnp.float32),
            pltpu.SemaphoreType.DMA,
        ],
        compiler_params=pltpu.CompilerParams(needs_layout_passes=False,
                                             use_tc_tiling_on_sc=False),
    )
    return fn(soft2, idx2, fp_sc, cb32)


def kernel(f_a, position_embedding, WG_w, WG_b, WK_w, WK_b, WQ_w, WQ_b,
           conv_w, conv_b):
    f_p = jnp.transpose(f_a, (1, 0, 2))
    wq, wk, fp = _proj_call(f_p, WQ_w, WQ_b, WK_w, WK_b, conv_w)
    b8 = jnp.tile(WG_b, (8,)).reshape(1, 128)
    spread = jnp.repeat(jnp.eye(8, dtype=jnp.float32), 16, axis=1)
    msk = jnp.tile(jnp.repeat(jnp.eye(_G, dtype=jnp.float32), _DG, axis=0),
                   (1, 8))
    soft, idx = _attn_call(position_embedding, wq, wk,
                           jnp.transpose(WG_w), b8, spread, msk)
    soft2 = jnp.transpose(soft.reshape(_C, _NG, 16, 8, _G),
                          (0, 4, 1, 3, 2)).reshape(_C * _G * _N, 16)
    idx2 = jnp.transpose(idx.reshape(_C, _NG, 16, 8, _G),
                         (0, 4, 1, 3, 2)).reshape(_C * _G * _N, 16)
    fp_sc = fp.reshape(_C * _N * _G, _DG)
    cbr = conv_b.reshape(_G, _DG)
    cb32 = jnp.concatenate([cbr, cbr], axis=0)
    out_sc = _sc_call(soft2, idx2, fp_sc, cb32)
    final = out_sc.reshape(_C, _G, _N, _DG)
    return jnp.transpose(final, (2, 0, 1, 3)).reshape(_N, _C, _G * _DG)


# speedup vs baseline: 1.3128x; 1.3128x over previous
import functools

import jax
import jax.numpy as jnp
from jax import lax
from jax.experimental import pallas as pl
from jax.experimental.pallas import tpu as pltpu
from jax.experimental.pallas import tpu_sc as plsc

_N = 1000
_C = 2
_FEAT = 1024
_G = 16
_GEO = 64
_DG = 64
_K = 10
_NB = 8
_MC = 1000


def _proj_body(f_ref, wqw_ref, wqb_ref, wkw_ref, wkb_ref, cw_ref,
               wq_ref, wk_ref, fp_ref):
    fc = f_ref[0]
    dn = (((1,), (1,)), ((), ()))
    wq = lax.dot_general(fc, wqw_ref[...], dn, preferred_element_type=jnp.float32)
    wq_ref[0] = wq + wqb_ref[...]
    wk = lax.dot_general(fc, wkw_ref[...], dn, preferred_element_type=jnp.float32)
    wk_ref[0] = wk + wkb_ref[...]
    fp_ref[0] = lax.dot_general(fc, cw_ref[...], dn, preferred_element_type=jnp.float32)


def _proj_call(f_p, WQ_w, WQ_b, WK_w, WK_b, conv_w):
    full = lambda *shape: pl.BlockSpec(shape, lambda c: (0,) * len(shape))
    return pl.pallas_call(
        _proj_body,
        grid=(_C,),
        in_specs=[
            pl.BlockSpec((1, _N, _FEAT), lambda c: (c, 0, 0)),
            full(_FEAT, _FEAT),
            full(1, _FEAT),
            full(_FEAT, _FEAT),
            full(1, _FEAT),
            full(_FEAT, _FEAT),
        ],
        out_specs=[
            pl.BlockSpec((1, _N, _FEAT), lambda c: (c, 0, 0)),
            pl.BlockSpec((1, _N, _FEAT), lambda c: (c, 0, 0)),
            pl.BlockSpec((1, _N, _FEAT), lambda c: (c, 0, 0)),
        ],
        out_shape=[
            jax.ShapeDtypeStruct((_C, _N, _FEAT), jnp.float32),
            jax.ShapeDtypeStruct((_C, _N, _FEAT), jnp.float32),
            jax.ShapeDtypeStruct((_C, _N, _FEAT), jnp.float32),
        ],
        compiler_params=pltpu.CompilerParams(vmem_limit_bytes=100 * 2**20),
    )(f_p, WQ_w, WQ_b.reshape(1, _FEAT), WK_w, WK_b.reshape(1, _FEAT), conv_w)


_NBL = 5
_NG = _N // 8


def _attn_body(pe_ref, wq_ref, wk_ref, w8_ref, b8_ref, spr_ref, msk_ref,
               soft_ref, idx_ref):
    wk = wk_ref[0]
    row = lax.broadcasted_iota(jnp.int32, (_N, 128), 0)
    big = jnp.int32(2**30)
    for i in range(_NBL):
        pe2 = pe_ref[0, i]
        P = jnp.dot(pe2, w8_ref[...], preferred_element_type=jnp.float32)
        L = jnp.log(jnp.maximum(P + b8_ref[...], 1e-6))
        wqb = wq_ref[0, i * 8:(i + 1) * 8, :]
        wqS = jnp.dot(jnp.transpose(wqb), spr_ref[...],
                      preferred_element_type=jnp.float32) * msk_ref[...]
        A = jnp.dot(wk, wqS, preferred_element_type=jnp.float32) * 0.125
        X = A + L
        vals, idxs = [], []
        for _ in range(_K):
            mx = jnp.max(X, axis=0, keepdims=True)
            am = jnp.min(jnp.where(X == mx, row, big), axis=0, keepdims=True)
            vals.append(mx)
            idxs.append(am)
            X = jnp.where(row == am, -jnp.inf, X)
        V = jnp.concatenate(vals, axis=0)
        E = jnp.exp(V - V[0:1])
        S = E / jnp.sum(E, axis=0, keepdims=True)
        soft_ref[0, i] = jnp.concatenate(
            [S, jnp.zeros((16 - _K, 128), jnp.float32)], axis=0)
        idx_ref[0, i] = jnp.concatenate(
            idxs + [jnp.zeros((16 - _K, 128), jnp.int32)], axis=0)


def _attn_call(pe_t, wq, wk, W8, b8, spread, msk):
    return pl.pallas_call(
        _attn_body,
        grid=(_C, _NG // _NBL),
        in_specs=[
            pl.BlockSpec((1, _NBL, _N, 512), lambda c, nb: (c, nb, 0, 0)),
            pl.BlockSpec((1, 8 * _NBL, _FEAT), lambda c, nb: (c, nb, 0)),
            pl.BlockSpec((1, _N, _FEAT), lambda c, nb: (c, 0, 0)),
            pl.BlockSpec((512, 128), lambda c, nb: (0, 0)),
            pl.BlockSpec((1, 128), lambda c, nb: (0, 0)),
            pl.BlockSpec((8, 128), lambda c, nb: (0, 0)),
            pl.BlockSpec((_FEAT, 128), lambda c, nb: (0, 0)),
        ],
        out_specs=[
            pl.BlockSpec((1, _NBL, 16, 128), lambda c, nb: (c, nb, 0, 0)),
            pl.BlockSpec((1, _NBL, 16, 128), lambda c, nb: (c, nb, 0, 0)),
        ],
        out_shape=[
            jax.ShapeDtypeStruct((_C, _NG, 16, 128), jnp.float32),
            jax.ShapeDtypeStruct((_C, _NG, 16, 128), jnp.int32),
        ],
        compiler_params=pltpu.CompilerParams(vmem_limit_bytes=100 * 2**20),
    )(pe_t, wq, wk, W8, b8, spread, msk)


_CH = 8


def _sc_body(soft_hbm, idx_hbm, fp_hbm, cb_hbm, out_hbm,
             soft_v, idx_v, rid_v, rows_v, cb_v, outc_v, sem):
    nc = 2
    b = lax.axis_index("s") * nc + lax.axis_index("c")
    c = b // _G
    g = b % _G
    base = b * _N

    pltpu.sync_copy(soft_hbm.at[pl.ds(base, _N)], soft_v)
    pltpu.sync_copy(idx_hbm.at[pl.ds(base, _N)], idx_v)
    pltpu.sync_copy(cb_hbm.at[b], cb_v)

    off = c * (_G * _N) + g

    def rbody(i, carry):
        rid_v[i // _CH, pl.ds((i % _CH) * 16, 16)] = idx_v[i] * 16 + off
        return carry

    lax.fori_loop(0, _N, rbody, 0)

    def cbody(ch, carry):
        pltpu.async_copy(fp_hbm.at[rid_v.at[ch]], rows_v, sem).wait()
        for j in range(_CH):
            it = ch * _CH + j
            isp = jnp.full((16,), it, jnp.int32)
            accs = [cb_v[q * 16:(q + 1) * 16] for q in range(4)]
            for k in range(_K):
                ksp = jnp.full((16,), k, jnp.int32)
                w = plsc.load_gather(soft_v, [isp, ksp])
                r = j * 16 + k
                for q in range(4):
                    accs[q] = accs[q] + w * rows_v[r, q * 16:(q + 1) * 16]
            for q in range(4):
                outc_v[j, q * 16:(q + 1) * 16] = accs[q]
        pltpu.sync_copy(outc_v, out_hbm.at[pl.ds(base + ch * _CH, _CH)])
        return carry

    lax.fori_loop(0, _N // _CH, cbody, 0)


def _sc_call(soft2, idx2, fp_sc, cb32):
    mesh = plsc.VectorSubcoreMesh(core_axis_name="c", subcore_axis_name="s")
    fn = pl.kernel(
        _sc_body,
        out_type=jax.ShapeDtypeStruct((_C * _G * _N, _DG), jnp.float32),
        mesh=mesh,
        scratch_types=[
            pltpu.VMEM((_N, 16), jnp.float32),
            pltpu.VMEM((_N, 16), jnp.int32),
            pltpu.VMEM((_N // _CH, _CH * 16), jnp.int32),
            pltpu.VMEM((_CH * 16, _DG), jnp.float32),
            pltpu.VMEM((_DG,), jnp.float32),
            pltpu.VMEM((_CH, _DG), jnp.float32),
            pltpu.SemaphoreType.DMA,
        ],
        compiler_params=pltpu.CompilerParams(needs_layout_passes=False,
                                             use_tc_tiling_on_sc=False),
    )
    return fn(soft2, idx2, fp_sc, cb32)


def kernel(f_a, position_embedding, WG_w, WG_b, WK_w, WK_b, WQ_w, WQ_b,
           conv_w, conv_b):
    f_p = jnp.transpose(f_a, (1, 0, 2))
    wq, wk, fp = _proj_call(f_p, WQ_w, WQ_b, WK_w, WK_b, conv_w)
    pe_t = position_embedding.reshape(_C, _NG, 8, _N, _GEO)
    pe_t = jnp.transpose(pe_t, (0, 1, 3, 2, 4)).reshape(_C, _NG, _N, 512)
    eye8 = jnp.eye(8, dtype=jnp.float32)
    W8 = (eye8[:, None, :, None] * jnp.transpose(WG_w)[None, :, None, :]
          ).reshape(512, 128)
    b8 = jnp.tile(WG_b, (8,)).reshape(1, 128)
    spread = jnp.repeat(eye8, 16, axis=1)
    msk = jnp.tile(jnp.repeat(jnp.eye(_G, dtype=jnp.float32), _DG, axis=0),
                   (1, 8))
    soft, idx = _attn_call(pe_t, wq, wk, W8, b8, spread, msk)
    soft2 = jnp.transpose(soft.reshape(_C, _NG, 16, 8, _G),
                          (0, 4, 1, 3, 2)).reshape(_C * _G * _N, 16)
    idx2 = jnp.transpose(idx.reshape(_C, _NG, 16, 8, _G),
                         (0, 4, 1, 3, 2)).reshape(_C * _G * _N, 16)
    fp_sc = fp.reshape(_C * _N * _G, _DG)
    cbr = conv_b.reshape(_G, _DG)
    cb32 = jnp.concatenate([cbr, cbr], axis=0)
    out_sc = _sc_call(soft2, idx2, fp_sc, cb32)
    final = out_sc.reshape(_C, _G, _N, _DG)
    return jnp.transpose(final, (2, 0, 1, 3)).reshape(_N, _C, _G * _DG)
